# baseline (speedup 1.0000x reference)
import functools

import numpy as np
import jax
import jax.numpy as jnp
from jax import lax
from jax.experimental import pallas as pl
from jax.experimental.pallas import tpu as pltpu
from jax.experimental.pallas import tpu_sc as plsc

NC = 2
NS = 16
NW = NC * NS
LANES = 16


def _pos_encoding(max_len, d):
    pos = np.arange(max_len, dtype=np.float32)[:, None]
    i = np.arange(0, d, 2, dtype=np.float32)
    angle = pos / np.power(10000.0, i / d)
    pe = np.zeros((max_len, d), dtype=np.float32)
    pe[:, 0::2] = np.sin(angle)
    pe[:, 1::2] = np.cos(angle)
    return pe


def _pe_packed(S, E):
    pe = _pos_encoding(S, E)
    u = pe.view(np.uint32)
    bf = ((u + 0x7FFF + ((u >> 16) & 1)) >> 16).astype(np.uint32)
    v = bf.reshape(S, E // (2 * LANES), 2, LANES)
    words = (v[:, :, 0, :] | (v[:, :, 1, :] << 16)).reshape(S, E // 2)
    return jnp.asarray(words.view(np.int32))


@functools.lru_cache(maxsize=None)
def _build(B, S, E, CP):
    assert S % NW == 0
    assert B % 2 == 0
    p_per_w = S // NW
    assert p_per_w % CP == 0
    NG = p_per_w // CP
    assert NG % 2 == 0
    nlc = E // LANES

    mesh = plsc.VectorSubcoreMesh(core_axis_name="c", subcore_axis_name="s")

    @functools.partial(
        pl.kernel,
        mesh=mesh,
        out_type=jax.ShapeDtypeStruct((B, S, E), jnp.float32),
        scratch_types=[
            pltpu.VMEM((B, p_per_w), jnp.int32),
            pltpu.VMEM((B, CP, E), jnp.float32),
            pltpu.VMEM((B, CP, E), jnp.float32),
            pltpu.VMEM((CP, E // 2), jnp.int32),
            pltpu.VMEM((CP, E // 2), jnp.int32),
            pltpu.SemaphoreType.DMA,
            pltpu.SemaphoreType.DMA,
            pltpu.SemaphoreType.DMA,
            pltpu.SemaphoreType.DMA,
            pltpu.SemaphoreType.DMA,
            pltpu.SemaphoreType.DMA,
        ],
    )
    def k(x_hbm, table_hbm, pe_hbm, out_hbm,
          idx_all, bf0, bf1, pe0, pe1, g0, g1, s0, s1, q0, q1):
        ci = lax.axis_index("c")
        si = lax.axis_index("s")
        wid = si * NC + ci
        p0 = wid * p_per_w

        bufs = [bf0, bf1]
        pes = [pe0, pe1]
        gsem = [g0, g1]
        ssem = [s0, s1]
        psem = [q0, q1]

        def issue_gathers(g, slot):
            for b in range(B):
                pltpu.async_copy(
                    table_hbm.at[idx_all.at[b, pl.ds(g * CP, CP)]],
                    bufs[slot].at[b], gsem[slot])

        def issue_pe(g, slot):
            pltpu.async_copy(pe_hbm.at[pl.ds(p0 + g * CP, CP)],
                             pes[slot], psem[slot])

        def wait_gather(g, slot, b):
            pltpu.make_async_copy(
                table_hbm.at[idx_all.at[b, pl.ds(g * CP, CP)]],
                bufs[slot].at[b], gsem[slot]).wait()

        def wait_stores(g, slot):
            for b in range(B):
                pltpu.make_async_copy(
                    bufs[slot].at[b],
                    out_hbm.at[b, pl.ds(p0 + g * CP, CP)],
                    ssem[slot]).wait()

        for b in range(B):
            pltpu.sync_copy(x_hbm.at[b, pl.ds(p0, p_per_w)],
                            idx_all.at[b])

        issue_gathers(0, 0)
        issue_pe(0, 0)

        def do_group(g, s):
            o = 1 - s
            pbase = p0 + g * CP
            gn = jnp.minimum(g + 1, NG - 1)
            issue_pe(gn, o)
            if s == 0:
                pl.when(g > 0)(lambda: wait_stores(g, o))
            else:
                wait_stores(g, o)
            issue_gathers(gn, o)
            pltpu.make_async_copy(pe_hbm.at[pl.ds(pbase, CP)],
                                  pes[s], psem[s]).wait()

            for h in range(B // 2):
                b0, b1 = 2 * h, 2 * h + 1
                wait_gather(g, s, b0)
                wait_gather(g, s, b1)

                @plsc.parallel_loop(0, CP, step=1, unroll=4)
                def add_body(r):
                    for j in range(nlc // 2):
                        col = j * 2 * LANES
                        w = pes[s][r, pl.ds(j * LANES, LANES)]
                        pa = lax.bitcast_convert_type(w << 16,
                                                      jnp.float32)
                        pb = lax.bitcast_convert_type(
                            w & jnp.int32(-(1 << 16)), jnp.float32)
                        for b in (b0, b1):
                            bufs[s][b, r, pl.ds(col, LANES)] = (
                                bufs[s][b, r, pl.ds(col, LANES)] + pa)
                            bufs[s][b, r, pl.ds(col + LANES, LANES)] = (
                                bufs[s][b, r, pl.ds(col + LANES, LANES)]
                                + pb)

                for b in (b0, b1):
                    pltpu.async_copy(bufs[s].at[b],
                                     out_hbm.at[b, pl.ds(pbase, CP)],
                                     ssem[s])

        @functools.partial(lax.fori_loop, 0, NG // 2, init_val=0)
        def _loop(gg, carry):
            do_group(2 * gg, 0)
            do_group(2 * gg + 1, 1)
            return carry

        wait_stores(NG - 1, 1)
        for b in range(B):
            pltpu.make_async_copy(
                table_hbm.at[idx_all.at[b, pl.ds((NG - 1) * CP, CP)]],
                bufs[0].at[b], gsem[0]).wait()
        pltpu.make_async_copy(pe_hbm.at[pl.ds(p0 + (NG - 1) * CP, CP)],
                              pes[0], psem[0]).wait()

    return k


def kernel(x, table):
    B, S = x.shape
    E = table.shape[1]
    CP = 8
    pe = _pe_packed(S, E)
    return _build(B, S, E, CP)(x, table, pe)


# speedup vs baseline: 1.3119x; 1.3119x over previous
import functools

import numpy as np
import jax
import jax.numpy as jnp
from jax import lax
from jax.experimental import pallas as pl
from jax.experimental.pallas import tpu as pltpu
from jax.experimental.pallas import tpu_sc as plsc

NC = 2
NS = 16
NW = NC * NS
LANES = 16


def _pos_encoding(max_len, d):
    pos = np.arange(max_len, dtype=np.float32)[:, None]
    i = np.arange(0, d, 2, dtype=np.float32)
    angle = pos / np.power(10000.0, i / d)
    pe = np.zeros((max_len, d), dtype=np.float32)
    pe[:, 0::2] = np.sin(angle)
    pe[:, 1::2] = np.cos(angle)
    return pe


def _pe_packed(S, E):
    pe = _pos_encoding(S, E)
    u = pe.view(np.uint32)
    bf = ((u + 0x7FFF + ((u >> 16) & 1)) >> 16).astype(np.uint32)
    v = bf.reshape(S, E // (2 * LANES), 2, LANES)
    words = (v[:, :, 0, :] | (v[:, :, 1, :] << 16)).reshape(S, E // 2)
    return jnp.asarray(words.view(np.int32))


@functools.lru_cache(maxsize=None)
def _build(B, S, E, CP):
    assert S % NW == 0
    assert B == 4 and CP == LANES
    p_per_w = S // NW
    assert p_per_w % CP == 0
    NCH = p_per_w // CP
    assert NCH % 2 == 0
    NGR = NCH * 2
    nlc = E // LANES
    GR = 2 * CP

    mesh = plsc.VectorSubcoreMesh(core_axis_name="c", subcore_axis_name="s")

    @functools.partial(
        pl.kernel,
        mesh=mesh,
        out_type=jax.ShapeDtypeStruct((B, S, E), jnp.float32),
        scratch_types=[
            pltpu.VMEM((B, p_per_w), jnp.int32),
            pltpu.VMEM((NGR, GR), jnp.int32),
            pltpu.VMEM((GR, E), jnp.float32),
            pltpu.VMEM((GR, E), jnp.float32),
            pltpu.VMEM((CP, E // 2), jnp.int32),
            pltpu.VMEM((CP, E // 2), jnp.int32),
            pltpu.SemaphoreType.DMA,
            pltpu.SemaphoreType.DMA,
            pltpu.SemaphoreType.DMA,
            pltpu.SemaphoreType.DMA,
            pltpu.SemaphoreType.DMA,
            pltpu.SemaphoreType.DMA,
        ],
    )
    def k(x_hbm, table_hbm, pe_hbm, out_hbm,
          idx_all, idx_gm, bf0, bf1, pe0, pe1, g0, g1, s0, s1, q0, q1):
        ci = lax.axis_index("c")
        si = lax.axis_index("s")
        wid = si * NC + ci
        p0 = wid * p_per_w

        bufs = [bf0, bf1]
        pes = [pe0, pe1]
        gsem = [g0, g1]
        ssem = [s0, s1]
        psem = [q0, q1]

        def issue_gather(g, slot):
            pltpu.async_copy(table_hbm.at[idx_gm.at[g]],
                             bufs[slot], gsem[slot])

        def wait_gather(g, slot):
            pltpu.make_async_copy(table_hbm.at[idx_gm.at[g]],
                                  bufs[slot], gsem[slot]).wait()

        def issue_pe(c, slot):
            pltpu.async_copy(pe_hbm.at[pl.ds(p0 + c * CP, CP)],
                             pes[slot], psem[slot])

        def wait_pe(c, slot):
            pltpu.make_async_copy(pe_hbm.at[pl.ds(p0 + c * CP, CP)],
                                  pes[slot], psem[slot]).wait()

        def wait_stores(g, slot):
            c = g // 2
            p = g % 2
            for hb in range(2):
                pltpu.make_async_copy(
                    bufs[slot].at[pl.ds(hb * CP, CP)],
                    out_hbm.at[2 * p + hb, pl.ds(p0 + c * CP, CP)],
                    ssem[slot]).wait()

        for b in range(B):
            pltpu.sync_copy(x_hbm.at[b, pl.ds(p0, p_per_w)],
                            idx_all.at[b])
        for c in range(NCH):
            for p in range(2):
                for hb in range(2):
                    idx_gm[2 * c + p, pl.ds(hb * CP, CP)] = (
                        idx_all[2 * p + hb, pl.ds(c * CP, CP)])

        issue_gather(0, 0)
        issue_pe(0, 0)

        def do_group(g, k_):
            s = k_ % 2
            o = 1 - s
            cdyn = g // 2
            pdyn = g % 2
            pbase = p0 + cdyn * CP
            if k_ == 0:
                issue_pe(jnp.minimum(cdyn + 1, NCH - 1), 1)
            elif k_ == 2:
                issue_pe(jnp.minimum(cdyn + 1, NCH - 1), 0)
            gn = jnp.minimum(g + 1, NGR - 1)
            if k_ == 0:
                pl.when(g > 0)(lambda: wait_stores(g, o))
            else:
                wait_stores(g, o)
            issue_gather(gn, o)
            if k_ in (0, 2):
                wait_pe(cdyn, (k_ // 2) % 2)
            wait_gather(g, s)

            ps = (k_ // 2) % 2

            @plsc.parallel_loop(0, CP, step=1, unroll=1)
            def add_body(r):
                for j in range(nlc // 2):
                    col = j * 2 * LANES
                    w = pes[ps][r, pl.ds(j * LANES, LANES)]
                    pa = lax.bitcast_convert_type(w << 16, jnp.float32)
                    pb = lax.bitcast_convert_type(
                        w & jnp.int32(-(1 << 16)), jnp.float32)
                    for hb in range(2):
                        row = hb * CP + r
                        bufs[s][row, pl.ds(col, LANES)] = (
                            bufs[s][row, pl.ds(col, LANES)] + pa)
                        bufs[s][row, pl.ds(col + LANES, LANES)] = (
                            bufs[s][row, pl.ds(col + LANES, LANES)] + pb)

            for hb in range(2):
                pltpu.async_copy(
                    bufs[s].at[pl.ds(hb * CP, CP)],
                    out_hbm.at[2 * pdyn + hb, pl.ds(pbase, CP)],
                    ssem[s])

        @functools.partial(lax.fori_loop, 0, NCH // 2, init_val=0)
        def _loop(t, carry):
            g0_ = 4 * t
            for k_ in range(4):
                do_group(g0_ + k_, k_)
            return carry

        wait_stores(NGR - 1, 1)
        pltpu.make_async_copy(table_hbm.at[idx_gm.at[NGR - 1]],
                              bufs[0], gsem[0]).wait()
        wait_pe(NCH - 1, 0)

    return k


def kernel(x, table):
    B, S = x.shape
    E = table.shape[1]
    CP = 16
    pe = _pe_packed(S, E)
    return _build(B, S, E, CP)(x, table, pe)


# speedup vs baseline: 1.5159x; 1.1555x over previous
import functools

import numpy as np
import jax
import jax.numpy as jnp
from jax import lax
from jax.experimental import pallas as pl
from jax.experimental.pallas import tpu as pltpu
from jax.experimental.pallas import tpu_sc as plsc

NC = 2
NS = 16
NW = NC * NS
LANES = 16


def _pos_encoding(max_len, d):
    pos = np.arange(max_len, dtype=np.float32)[:, None]
    i = np.arange(0, d, 2, dtype=np.float32)
    angle = pos / np.power(10000.0, i / d)
    pe = np.zeros((max_len, d), dtype=np.float32)
    pe[:, 0::2] = np.sin(angle)
    pe[:, 1::2] = np.cos(angle)
    return pe


def _pe_packed(S, E):
    pe = _pos_encoding(S, E)
    u = pe.view(np.uint32)
    bf = ((u + 0x7FFF + ((u >> 16) & 1)) >> 16).astype(np.uint32)
    v = bf.reshape(S, E // (2 * LANES), 2, LANES)
    words = (v[:, :, 0, :] | (v[:, :, 1, :] << 16)).reshape(S, E // 2)
    return jnp.asarray(words.view(np.int32))


@functools.lru_cache(maxsize=None)
def _build(B, S, E, CP):
    assert S % NW == 0
    assert B % 2 == 0
    p_per_w = S // NW
    assert p_per_w % CP == 0
    NG = p_per_w // CP
    assert NG % 2 == 0
    nlc = E // LANES

    mesh = plsc.VectorSubcoreMesh(core_axis_name="c", subcore_axis_name="s")

    @functools.partial(
        pl.kernel,
        mesh=mesh,
        out_type=jax.ShapeDtypeStruct((B, S, E), jnp.float32),
        scratch_types=[
            pltpu.VMEM((B, p_per_w), jnp.int32),
            pltpu.VMEM((B, CP, E), jnp.float32),
            pltpu.VMEM((B, CP, E), jnp.float32),
            pltpu.VMEM((CP, E // 2), jnp.int32),
            pltpu.VMEM((CP, E // 2), jnp.int32),
            pltpu.SemaphoreType.DMA,
            pltpu.SemaphoreType.DMA,
            pltpu.SemaphoreType.DMA,
            pltpu.SemaphoreType.DMA,
            pltpu.SemaphoreType.DMA,
            pltpu.SemaphoreType.DMA,
        ],
    )
    def k(x_hbm, table_hbm, pe_hbm, out_hbm,
          idx_all, bf0, bf1, pe0, pe1, g0, g1, s0, s1, q0, q1):
        ci = lax.axis_index("c")
        si = lax.axis_index("s")
        wid = si * NC + ci
        p0 = wid * p_per_w

        bufs = [bf0, bf1]
        pes = [pe0, pe1]
        gsem = [g0, g1]
        ssem = [s0, s1]
        psem = [q0, q1]

        def issue_gathers(g, slot):
            for b in range(B):
                pltpu.async_copy(
                    table_hbm.at[idx_all.at[b, pl.ds(g * CP, CP)]],
                    bufs[slot].at[b], gsem[slot])

        def issue_pe(g, slot):
            pltpu.async_copy(pe_hbm.at[pl.ds(p0 + g * CP, CP)],
                             pes[slot], psem[slot])

        def wait_gather(g, slot, b):
            pltpu.make_async_copy(
                table_hbm.at[idx_all.at[b, pl.ds(g * CP, CP)]],
                bufs[slot].at[b], gsem[slot]).wait()

        def wait_stores(g, slot):
            for b in range(B):
                pltpu.make_async_copy(
                    bufs[slot].at[b],
                    out_hbm.at[b, pl.ds(p0 + g * CP, CP)],
                    ssem[slot]).wait()

        pltpu.sync_copy(x_hbm.at[:, pl.ds(p0, p_per_w)], idx_all)

        issue_gathers(0, 0)
        issue_pe(0, 0)

        def do_group(g, s):
            o = 1 - s
            pbase = p0 + g * CP
            gn = jnp.minimum(g + 1, NG - 1)
            issue_pe(gn, o)
            if s == 0:
                pl.when(g > 0)(lambda: wait_stores(g, o))
            else:
                wait_stores(g, o)
            issue_gathers(gn, o)
            pltpu.make_async_copy(pe_hbm.at[pl.ds(pbase, CP)],
                                  pes[s], psem[s]).wait()

            for h in range(B // 2):
                b0, b1 = 2 * h, 2 * h + 1
                wait_gather(g, s, b0)
                wait_gather(g, s, b1)

                @plsc.parallel_loop(0, CP, step=1, unroll=2)
                def add_body(r):
                    for j in range(nlc // 2):
                        col = j * 2 * LANES
                        w = pes[s][r, pl.ds(j * LANES, LANES)]
                        pa = lax.bitcast_convert_type(w << 16,
                                                      jnp.float32)
                        pb = lax.bitcast_convert_type(
                            w & jnp.int32(-(1 << 16)), jnp.float32)
                        for b in (b0, b1):
                            bufs[s][b, r, pl.ds(col, LANES)] = (
                                bufs[s][b, r, pl.ds(col, LANES)] + pa)
                            bufs[s][b, r, pl.ds(col + LANES, LANES)] = (
                                bufs[s][b, r, pl.ds(col + LANES, LANES)]
                                + pb)

                for b in (b0, b1):
                    pltpu.async_copy(bufs[s].at[b],
                                     out_hbm.at[b, pl.ds(pbase, CP)],
                                     ssem[s])

        @functools.partial(lax.fori_loop, 0, NG // 2, init_val=0)
        def _loop(gg, carry):
            do_group(2 * gg, 0)
            do_group(2 * gg + 1, 1)
            return carry

        wait_stores(NG - 1, 1)
        for b in range(B):
            pltpu.make_async_copy(
                table_hbm.at[idx_all.at[b, pl.ds((NG - 1) * CP, CP)]],
                bufs[0].at[b], gsem[0]).wait()
        pltpu.make_async_copy(pe_hbm.at[pl.ds(p0 + (NG - 1) * CP, CP)],
                              pes[0], psem[0]).wait()

    return k


def kernel(x, table):
    B, S = x.shape
    E = table.shape[1]
    CP = 8
    pe = _pe_packed(S, E)
    return _build(B, S, E, CP)(x, table, pe)
